# baseline (speedup 1.0000x reference)
import functools

import jax
import jax.numpy as jnp
from jax import lax
from jax.experimental import pallas as pl
from jax.experimental.pallas import tpu as pltpu
from jax.experimental.pallas import tpu_sc as plsc

_NUM_CLASSES = 1000000
_HIDDEN = 64
_DROPOUT_PROB = 0.1
_CHUNK = 128
_LANES = 16


@functools.lru_cache(maxsize=None)
def _build_sc_gather(batch: int, hidden: int):
    info = plsc.get_sparse_core_info()
    nc, ns = info.num_cores, info.num_subcores
    nw = nc * ns
    b_per_w = batch // nw
    n_chunks = b_per_w // _CHUNK

    mesh = plsc.VectorSubcoreMesh(core_axis_name="c", subcore_axis_name="s")

    @functools.partial(
        pl.kernel,
        out_type=jax.ShapeDtypeStruct((batch, hidden), jnp.float32),
        mesh=mesh,
        compiler_params=pltpu.CompilerParams(use_tc_tiling_on_sc=False),
        scratch_types=[
            pltpu.VMEM((n_chunks, _CHUNK), jnp.int32),
            pltpu.VMEM((n_chunks, _CHUNK), jnp.int32),
            pltpu.VMEM((b_per_w, hidden), jnp.float32),
            pltpu.SemaphoreType.DMA,
        ],
    )
    def k(idx_hbm, drop_hbm, table_hbm, out_hbm, idx_v, drop_v, rows_v, sem):
        wid = lax.axis_index("s") * nc + lax.axis_index("c")
        row0 = wid * n_chunks
        pltpu.sync_copy(idx_hbm.at[pl.ds(row0, n_chunks)], idx_v)
        pltpu.sync_copy(drop_hbm.at[pl.ds(row0, n_chunks)], drop_v)
        for j in range(n_chunks):
            for t in range(_CHUNK // _LANES):
                sl = pl.ds(t * _LANES, _LANES)
                lab = idx_v[j, sl]
                dr = drop_v[j, sl]
                idx_v[j, sl] = jnp.where(
                    dr != 0, jnp.full((_LANES,), _NUM_CLASSES, jnp.int32), lab)
        copies = [
            pltpu.async_copy(
                table_hbm.at[idx_v.at[j]],
                rows_v.at[pl.ds(j * _CHUNK, _CHUNK)],
                sem,
            )
            for j in range(n_chunks)
        ]
        for c in copies:
            c.wait()
        base = wid * b_per_w
        pltpu.sync_copy(rows_v, out_hbm.at[pl.ds(base, b_per_w)])

    return k


def kernel(labels, table, train):
    batch = labels.shape[0]
    hidden = table.shape[1]
    drop_ids = jax.random.uniform(jax.random.key(42), (batch,)) < _DROPOUT_PROB
    gate = jnp.asarray(train) != 0
    drop = (drop_ids & gate).astype(jnp.int32).reshape(batch // _CHUNK, _CHUNK)
    idx = labels.astype(jnp.int32).reshape(batch // _CHUNK, _CHUNK)
    return _build_sc_gather(batch, hidden)(idx, drop, table)


# speedup vs baseline: 1.7130x; 1.7130x over previous
import functools

import jax
import jax.numpy as jnp
from jax import lax
from jax.experimental import pallas as pl
from jax.experimental.pallas import tpu as pltpu
from jax.experimental.pallas import tpu_sc as plsc

_NUM_CLASSES = 1000000
_DROPOUT_PROB = 0.1
_LANES = 16


@functools.lru_cache(maxsize=None)
def _build_sc_gather(batch: int, hidden: int):
    info = plsc.get_sparse_core_info()
    nc, ns = info.num_cores, info.num_subcores
    nw = nc * ns
    b_per_w = batch // nw

    mesh = plsc.VectorSubcoreMesh(core_axis_name="c", subcore_axis_name="s")

    @functools.partial(
        pl.kernel,
        out_type=jax.ShapeDtypeStruct((batch, hidden), jnp.float32),
        mesh=mesh,
        scratch_types=[
            pltpu.VMEM((b_per_w,), jnp.int32),
            pltpu.VMEM((b_per_w,), jnp.int32),
            pltpu.VMEM((b_per_w, hidden), jnp.float32),
            pltpu.SemaphoreType.DMA,
            pltpu.SemaphoreType.DMA,
        ],
    )
    def k(idx_hbm, drop_hbm, table_hbm, out_hbm, idx_v, drop_v, rows_v,
          gsem, osem):
        wid = lax.axis_index("s") * nc + lax.axis_index("c")
        base = wid * b_per_w
        pltpu.sync_copy(idx_hbm.at[pl.ds(base, b_per_w)], idx_v)
        pltpu.sync_copy(drop_hbm.at[pl.ds(base, b_per_w)], drop_v)
        for t in range(b_per_w // _LANES):
            sl = pl.ds(t * _LANES, _LANES)
            lab = idx_v[sl]
            dr = drop_v[sl]
            idx_v[sl] = jnp.where(
                dr != 0, jnp.full((_LANES,), _NUM_CLASSES, jnp.int32), lab)

        def fetch(g, _):
            vec = idx_v[pl.ds(g * _LANES, _LANES)]
            for e in range(_LANES):
                s = vec[e]
                pltpu.async_copy(table_hbm.at[pl.ds(s, 1)],
                                 rows_v.at[pl.ds(g * _LANES + e, 1)], gsem)
            return 0

        lax.fori_loop(0, b_per_w // _LANES, fetch, 0)
        pltpu.make_async_copy(
            table_hbm.at[pl.ds(0, b_per_w)], rows_v, gsem).wait()
        pltpu.async_copy(rows_v, out_hbm.at[pl.ds(base, b_per_w)], osem).wait()

    return k


def kernel(labels, table, train):
    batch = labels.shape[0]
    hidden = table.shape[1]
    drop_ids = jax.random.uniform(jax.random.key(42), (batch,)) < _DROPOUT_PROB
    gate = jnp.asarray(train) != 0
    drop = (drop_ids & gate).astype(jnp.int32)
    idx = labels.astype(jnp.int32)
    return _build_sc_gather(batch, hidden)(idx, drop, table)


# speedup vs baseline: 3.1424x; 1.8345x over previous
import functools

import jax
import jax.numpy as jnp
from jax import lax
from jax.experimental import pallas as pl
from jax.experimental.pallas import tpu as pltpu
from jax.experimental.pallas import tpu_sc as plsc

_NUM_CLASSES = 1000000
_DROPOUT_PROB = 0.1
_LANES = 16
_SLAB = 512
_RING = 32
_DRAIN = 16


@functools.lru_cache(maxsize=None)
def _build_sc_gather(batch: int, hidden: int, classes: int):
    info = plsc.get_sparse_core_info()
    nc, ns = info.num_cores, info.num_subcores
    nw = nc * ns
    b_per_w = batch // nw
    classes_pad = -(-classes // 128) * 128
    max_start = classes_pad - _SLAB

    mesh = plsc.VectorSubcoreMesh(core_axis_name="c", subcore_axis_name="s")

    @functools.partial(
        pl.kernel,
        out_type=jax.ShapeDtypeStruct((batch, hidden), jnp.float32),
        mesh=mesh,
        compiler_params=pltpu.CompilerParams(needs_layout_passes=False),
        scratch_types=[
            pltpu.VMEM((b_per_w + _LANES,), jnp.int32),
            pltpu.VMEM((b_per_w + _LANES,), jnp.int32),
            pltpu.VMEM((hidden, _SLAB), jnp.float32),
            pltpu.VMEM((_RING, hidden), jnp.float32),
            pltpu.SMEM((2,), jnp.int32),
            pltpu.SemaphoreType.DMA,
            pltpu.SemaphoreType.DMA,
        ],
    )
    def k(lab_hbm, pos_hbm, tab_t_hbm, out_hbm, lab_v, pos_v, slab_v,
          ring_v, smem, fsem, wsem):
        wid = lax.axis_index("s") * nc + lax.axis_index("c")
        base = wid * b_per_w
        pltpu.sync_copy(lab_hbm.at[pl.ds(base, b_per_w + _LANES)], lab_v)
        pltpu.sync_copy(pos_hbm.at[pl.ds(base, b_per_w + _LANES)], pos_v)

        row_ids = [lax.iota(jnp.int32, 16) + 16 * t for t in range(hidden // 16)]

        def ext(ref, p):
            return ref[pl.ds(p, _LANES)][0]

        smem[0] = jnp.int32(0)
        smem[1] = jnp.int32(-1)

        def body(p, _):
            lab = ext(lab_v, p)

            @pl.when(lab >= smem[1])
            def _():
                start = jnp.minimum((lab >> 7) << 7, max_start)
                start = pl.multiple_of(start, 128)
                pltpu.async_copy(
                    tab_t_hbm.at[:, pl.ds(start, _SLAB)], slab_v, fsem).wait()
                smem[0] = start
                smem[1] = start + _SLAB

            l = lab - smem[0]
            col = jnp.full((_LANES,), l, jnp.int32)
            slot = lax.rem(p, _RING)
            for t in range(hidden // 16):
                vec = plsc.load_gather(slab_v, [row_ids[t], col])
                ring_v[slot, pl.ds(t * 16, 16)] = vec
            pos = ext(pos_v, p)
            pltpu.async_copy(ring_v.at[pl.ds(slot, 1)],
                             out_hbm.at[pl.ds(pos, 1)], wsem)

            @pl.when(lax.rem(p, _RING) == _RING - 1)
            def _():
                pltpu.make_async_copy(
                    ring_v, out_hbm.at[pl.ds(0, _RING)], wsem).wait()

            return 0

        lax.fori_loop(0, b_per_w, body, 0)

    return k


def kernel(labels, table, train):
    batch = labels.shape[0]
    classes, hidden = table.shape
    drop_ids = jax.random.uniform(jax.random.key(42), (batch,)) < _DROPOUT_PROB
    gate = jnp.asarray(train) != 0
    masked = jnp.where(drop_ids & gate, classes - 1, labels.astype(jnp.int32))
    pos = lax.iota(jnp.int32, batch)
    sorted_lab, sorted_pos = lax.sort_key_val(masked, pos)
    sentinel = jnp.full((_LANES,), jnp.int32(2**30))
    sorted_lab = jnp.concatenate([sorted_lab, sentinel])
    sorted_pos = jnp.concatenate([sorted_pos, jnp.zeros((_LANES,), jnp.int32)])
    return _build_sc_gather(batch, hidden, classes)(
        sorted_lab, sorted_pos, table.T)
